# baseline (speedup 1.0000x reference)
import functools

import jax
import jax.numpy as jnp
from jax import lax
from jax.experimental import pallas as pl
from jax.experimental.pallas import tpu as pltpu
from jax.experimental.pallas import tpu_sc as plsc

N = 10000
E = 320000
IN_F = 128
H1F = 256
H2F = 128
NCLS = 47
D2 = 48
DH = 16
CHUNK = 128
NCHUNK = E // CHUNK
NSUB = 16
NW = 2 * NSUB
ITERS = (NCHUNK + NW - 1) // NW
ROWS_PER_SUB = 624
TAIL_ROWS = N - NSUB * ROWS_PER_SUB


def _mesh():
    return plsc.VectorSubcoreMesh(core_axis_name="c", subcore_axis_name="s")


_LINEAR = pltpu.CompilerParams(use_tc_tiling_on_sc=False)


RB = 104


@functools.partial(
    pl.kernel,
    mesh=_mesh(),
    compiler_params=_LINEAR,
    out_type=jax.ShapeDtypeStruct((2, 2, N, IN_F), jnp.float32),
    scratch_types=[
        pltpu.VMEM((CHUNK, DH), jnp.float32),
        pltpu.VMEM((1, CHUNK), jnp.int32),
        pltpu.VMEM((1, CHUNK), jnp.int32),
        pltpu.VMEM((1, CHUNK), jnp.int32),
        pltpu.VMEM((1, CHUNK), jnp.int32),
        pltpu.VMEM((ROWS_PER_SUB + TAIL_ROWS, DH), jnp.float32),
        pltpu.VMEM((RB, IN_F), jnp.float32),
        pltpu.VMEM_SHARED((N, DH), jnp.float32),
        pltpu.VMEM_SHARED((N, DH), jnp.float32),
        pltpu.SemaphoreType.DMA,
        pltpu.SemaphoreType.DMA,
        pltpu.SemaphoreType.DMA,
        pltpu.SemaphoreType.DMA,
    ],
)
def _deg_kernel(src_hbm, dst_hbm, ones_hbm, zeros_hbm, out_hbm,
                ones_v, sidx0, sidx1, didx0, didx1, slab_v, wide_v,
                acc_s, acc_d, sem0, sem1, ssem0, ssem1):
    cid = lax.axis_index("c")
    sid = lax.axis_index("s")
    wid = cid * NSUB + sid
    r0 = sid * ROWS_PER_SUB
    sidx = (sidx0, sidx1)
    didx = (didx0, didx1)
    sems = (sem0, sem1)
    ssem = (ssem0, ssem1)
    npair = (ITERS + 1) // 2

    def fire(slot, ordinal, first=False):
        if not first:
            cprev = wid + (ordinal - 2) * NW

            @pl.when(cprev < NCHUNK)
            def _():
                pltpu.make_async_copy(ones_v, acc_s.at[sidx[slot].at[0]],
                                      ssem[slot]).wait()
                pltpu.make_async_copy(ones_v, acc_d.at[didx[slot].at[0]],
                                      ssem[slot]).wait()

        c = wid + ordinal * NW

        @pl.when(c < NCHUNK)
        def _():
            base = c * CHUNK
            pltpu.async_copy(src_hbm.at[pl.ds(base, CHUNK)],
                             sidx[slot].at[0], sems[slot])
            pltpu.async_copy(dst_hbm.at[pl.ds(base, CHUNK)],
                             didx[slot].at[0], sems[slot])

    def drain(slot, ordinal):
        c = wid + ordinal * NW

        @pl.when(c < NCHUNK)
        def _():
            base = c * CHUNK
            pltpu.make_async_copy(src_hbm.at[pl.ds(base, CHUNK)],
                                  sidx[slot].at[0], sems[slot]).wait()
            pltpu.make_async_copy(dst_hbm.at[pl.ds(base, CHUNK)],
                                  didx[slot].at[0], sems[slot]).wait()
            pltpu.async_copy(ones_v, acc_s.at[sidx[slot].at[0]],
                             ssem[slot], add=True)
            pltpu.async_copy(ones_v, acc_d.at[didx[slot].at[0]],
                             ssem[slot], add=True)

    pltpu.sync_copy(ones_hbm, ones_v)
    pltpu.sync_copy(zeros_hbm.at[pl.ds(r0, ROWS_PER_SUB)],
                    acc_s.at[pl.ds(r0, ROWS_PER_SUB)])
    pltpu.sync_copy(zeros_hbm.at[pl.ds(r0, ROWS_PER_SUB)],
                    acc_d.at[pl.ds(r0, ROWS_PER_SUB)])

    @pl.when(sid == NSUB - 1)
    def _():
        pltpu.sync_copy(zeros_hbm.at[pl.ds(NSUB * ROWS_PER_SUB, TAIL_ROWS)],
                        acc_s.at[pl.ds(NSUB * ROWS_PER_SUB, TAIL_ROWS)])
        pltpu.sync_copy(zeros_hbm.at[pl.ds(NSUB * ROWS_PER_SUB, TAIL_ROWS)],
                        acc_d.at[pl.ds(NSUB * ROWS_PER_SUB, TAIL_ROWS)])

    plsc.subcore_barrier()

    fire(0, 0, first=True)
    fire(1, 1, first=True)

    @pl.loop(0, npair)
    def _(j):
        i0 = 2 * j
        drain(0, i0)
        fire(0, i0 + 2)
        drain(1, i0 + 1)
        fire(1, i0 + 3)

    plsc.subcore_barrier()

    def expand(acc, h):
        pltpu.sync_copy(acc.at[pl.ds(r0, ROWS_PER_SUB)],
                        slab_v.at[pl.ds(0, ROWS_PER_SUB)])

        @pl.when(sid == NSUB - 1)
        def _():
            pltpu.sync_copy(acc.at[pl.ds(NSUB * ROWS_PER_SUB, TAIL_ROWS)],
                            slab_v.at[pl.ds(ROWS_PER_SUB, TAIL_ROWS)])

        nblk = ROWS_PER_SUB // RB

        @pl.loop(0, nblk)
        def _(b):
            @pl.loop(0, RB)
            def _(r):
                v = slab_v[b * RB + r, :]
                for k in range(IN_F // DH):
                    wide_v[r, pl.ds(k * DH, DH)] = v

            pltpu.sync_copy(wide_v,
                            out_hbm.at[cid, h, pl.ds(r0 + b * RB, RB)])

        @pl.when(sid == NSUB - 1)
        def _():
            @pl.loop(0, TAIL_ROWS)
            def _(r):
                v = slab_v[ROWS_PER_SUB + r, :]
                for k in range(IN_F // DH):
                    wide_v[r, pl.ds(k * DH, DH)] = v

            pltpu.sync_copy(wide_v.at[pl.ds(0, TAIL_ROWS)],
                            out_hbm.at[cid, h,
                                       pl.ds(NSUB * ROWS_PER_SUB, TAIL_ROWS)])

    expand(acc_s, 0)
    expand(acc_d, 1)


def _make_agg(D, nslot, stage_feat=False):
    ngroup = (ITERS + nslot - 1) // nslot

    idx_scratch = [pltpu.VMEM((CHUNK,), jnp.int32) for _ in range(nslot)]
    didx_scratch = [pltpu.VMEM((1, CHUNK), jnp.int32) for _ in range(nslot)]
    row_scratch = [pltpu.VMEM((CHUNK, D), jnp.float32) for _ in range(nslot)]
    sem_scratch = [pltpu.SemaphoreType.DMA for _ in range(2 * nslot)]
    stage_scratch = (
        [pltpu.VMEM_SHARED((N, D), jnp.float32)] if stage_feat else [])

    @functools.partial(
        pl.kernel,
        mesh=_mesh(),
        compiler_params=_LINEAR,
        out_type=jax.ShapeDtypeStruct((2, N, D), jnp.float32),
        scratch_types=idx_scratch + didx_scratch + row_scratch
        + [pltpu.VMEM_SHARED((N, D), jnp.float32)] + sem_scratch
        + stage_scratch,
    )
    def agg(feat_hbm, src_hbm, dst_hbm, zeros_hbm, out_hbm, *scratch):
        sidx = scratch[0:nslot]
        didx = scratch[nslot:2 * nslot]
        rows = scratch[2 * nslot:3 * nslot]
        acc = scratch[3 * nslot]
        gsem = scratch[3 * nslot + 1:3 * nslot + 1 + nslot]
        ssem = scratch[3 * nslot + 1 + nslot:3 * nslot + 1 + 2 * nslot]
        cid = lax.axis_index("c")
        sid = lax.axis_index("s")
        wid = cid * NSUB + sid
        r0 = sid * ROWS_PER_SUB
        feat = scratch[3 * nslot + 1 + 2 * nslot] if stage_feat else feat_hbm

        def fire(slot, ordinal, first=False):
            if not first:
                cprev = wid + (ordinal - nslot) * NW

                @pl.when(cprev < NCHUNK)
                def _():
                    pltpu.make_async_copy(rows[slot],
                                          acc.at[didx[slot].at[0]],
                                          ssem[slot]).wait()

            c = wid + ordinal * NW

            @pl.when(c < NCHUNK)
            def _():
                base = c * CHUNK
                pltpu.sync_copy(src_hbm.at[pl.ds(base, CHUNK)], sidx[slot])
                pltpu.async_copy(feat.at[sidx[slot]], rows[slot], gsem[slot])
                pltpu.sync_copy(dst_hbm.at[pl.ds(base, CHUNK)], didx[slot].at[0])

        def drain(slot, ordinal):
            c = wid + ordinal * NW

            @pl.when(c < NCHUNK)
            def _():
                pltpu.make_async_copy(feat.at[sidx[slot]], rows[slot],
                                      gsem[slot]).wait()
                pltpu.async_copy(rows[slot], acc.at[didx[slot].at[0]],
                                 ssem[slot], add=True)

        pltpu.sync_copy(zeros_hbm.at[pl.ds(r0, ROWS_PER_SUB)],
                        acc.at[pl.ds(r0, ROWS_PER_SUB)])
        if stage_feat:
            pltpu.sync_copy(feat_hbm.at[pl.ds(r0, ROWS_PER_SUB)],
                            feat.at[pl.ds(r0, ROWS_PER_SUB)])

        @pl.when(sid == NSUB - 1)
        def _():
            pltpu.sync_copy(zeros_hbm.at[pl.ds(NSUB * ROWS_PER_SUB, TAIL_ROWS)],
                            acc.at[pl.ds(NSUB * ROWS_PER_SUB, TAIL_ROWS)])
            if stage_feat:
                pltpu.sync_copy(feat_hbm.at[pl.ds(NSUB * ROWS_PER_SUB,
                                                  TAIL_ROWS)],
                                feat.at[pl.ds(NSUB * ROWS_PER_SUB, TAIL_ROWS)])

        plsc.subcore_barrier()

        for s in range(nslot):
            fire(s, s, first=True)

        @pl.loop(0, ngroup)
        def _(j):
            i0 = j * nslot
            for s in range(nslot):
                drain(s, i0 + s)
                fire(s, i0 + s + nslot)

        plsc.subcore_barrier()
        pltpu.sync_copy(acc.at[pl.ds(r0, ROWS_PER_SUB)],
                        out_hbm.at[cid, pl.ds(r0, ROWS_PER_SUB)])

        @pl.when(sid == NSUB - 1)
        def _():
            pltpu.sync_copy(acc.at[pl.ds(NSUB * ROWS_PER_SUB, TAIL_ROWS)],
                            out_hbm.at[cid, pl.ds(NSUB * ROWS_PER_SUB, TAIL_ROWS)])

    return agg


_agg128 = _make_agg(IN_F, 3)
_agg48 = _make_agg(D2, 6, stage_feat=True)


BN = 2000


def _tc_norm_scale(x, deg_p):
    def body(x_ref, d_ref, xs_ref, nrm_ref):
        d = d_ref[...]
        ns = lax.rsqrt(jnp.maximum(d[0, 0] + d[1, 0], 1.0))
        nd = lax.rsqrt(jnp.maximum(d[0, 1] + d[1, 1], 1.0))
        nrm_ref[...] = jnp.concatenate([ns[:, :1], nd[:, :1]], axis=1)
        xs_ref[...] = x_ref[...] * ns

    return pl.pallas_call(
        body,
        grid=(N // BN,),
        in_specs=[
            pl.BlockSpec((BN, IN_F), lambda i: (i, 0)),
            pl.BlockSpec((2, 2, BN, IN_F), lambda i: (0, 0, i, 0)),
        ],
        out_specs=[
            pl.BlockSpec((BN, IN_F), lambda i: (i, 0)),
            pl.BlockSpec((BN, 2), lambda i: (i, 0)),
        ],
        out_shape=[
            jax.ShapeDtypeStruct((N, IN_F), jnp.float32),
            jax.ShapeDtypeStruct((N, 2), jnp.float32),
        ],
    )(x, deg_p)


def _tc_mid(p, nrm, w1, b1, w2, wfc_p):
    def body(p_ref, n_ref, w1_ref, b1_ref, w2_ref, wfc_ref, g_ref):
        nd = n_ref[:, 1]
        ns = n_ref[:, 0]
        a = (p_ref[0] + p_ref[1]) * nd[:, None]
        h = jnp.dot(a, w1_ref[...], preferred_element_type=jnp.float32)
        h = jnp.maximum(h + b1_ref[...][None, :], 0.0)
        wc = jnp.dot(w2_ref[...], wfc_ref[...],
                     preferred_element_type=jnp.float32)
        g_ref[...] = jnp.dot(h * ns[:, None], wc,
                             preferred_element_type=jnp.float32)

    return pl.pallas_call(
        body,
        grid=(N // BN,),
        in_specs=[
            pl.BlockSpec((2, BN, IN_F), lambda i: (0, i, 0)),
            pl.BlockSpec((BN, 2), lambda i: (i, 0)),
            pl.BlockSpec((IN_F, H1F), lambda i: (0, 0)),
            pl.BlockSpec((H1F,), lambda i: (0,)),
            pl.BlockSpec((H1F, H2F), lambda i: (0, 0)),
            pl.BlockSpec((H2F, D2), lambda i: (0, 0)),
        ],
        out_specs=pl.BlockSpec((BN, D2), lambda i: (i, 0)),
        out_shape=jax.ShapeDtypeStruct((N, D2), jnp.float32),
    )(p, nrm, w1, b1, w2, wfc_p)


def _tc_final(q, nrm, b2, wfc_p, bfc_p):
    def body(q_ref, n_ref, b2_ref, wfc_ref, bfc_ref, o_ref):
        bc = jnp.dot(b2_ref[...][None, :], wfc_ref[...],
                     preferred_element_type=jnp.float32)[0] + bfc_ref[...]
        nd = n_ref[:, 1]
        o_ref[...] = (q_ref[0] + q_ref[1]) * nd[:, None] + bc[None, :]

    return pl.pallas_call(
        body,
        grid=(N // BN,),
        in_specs=[
            pl.BlockSpec((2, BN, D2), lambda i: (0, i, 0)),
            pl.BlockSpec((BN, 2), lambda i: (i, 0)),
            pl.BlockSpec((H2F,), lambda i: (0,)),
            pl.BlockSpec((H2F, D2), lambda i: (0, 0)),
            pl.BlockSpec((D2,), lambda i: (0,)),
        ],
        out_specs=pl.BlockSpec((BN, D2), lambda i: (i, 0)),
        out_shape=jax.ShapeDtypeStruct((N, D2), jnp.float32),
    )(q, nrm, b2, wfc_p, bfc_p)


@jax.jit
def kernel(x, edge_index, W1, b1, W2, b2, Wfc, bfc):
    src = edge_index[0]
    dst = edge_index[1]
    ones16 = jnp.ones((CHUNK, DH), jnp.float32)
    zeros16 = jnp.zeros((N, DH), jnp.float32)
    zeros128 = jnp.zeros((N, IN_F), jnp.float32)
    zeros48 = jnp.zeros((N, D2), jnp.float32)
    wfc_p = jnp.pad(Wfc, ((0, 0), (0, D2 - NCLS)))
    bfc_p = jnp.pad(bfc, (0, D2 - NCLS))

    deg_p = _deg_kernel(src, dst, ones16, zeros16)
    xs, nrm = _tc_norm_scale(x, deg_p)
    p = _agg128(xs, src, dst, zeros128)
    g = _tc_mid(p, nrm, W1, b1, W2, wfc_p)
    q = _agg48(g, src, dst, zeros48)
    out = _tc_final(q, nrm, b2, wfc_p, bfc_p)
    return out[:, :NCLS]


# speedup vs baseline: 1.0360x; 1.0360x over previous
import functools

import jax
import jax.numpy as jnp
from jax import lax
from jax.experimental import pallas as pl
from jax.experimental.pallas import tpu as pltpu
from jax.experimental.pallas import tpu_sc as plsc

N = 10000
E = 320000
IN_F = 128
H1F = 256
H2F = 128
NCLS = 47
D2 = 48
DH = 16
CHUNK = 128
NCHUNK = E // CHUNK
NSUB = 16
NW = 2 * NSUB
ITERS = (NCHUNK + NW - 1) // NW
ROWS_PER_SUB = 624
TAIL_ROWS = N - NSUB * ROWS_PER_SUB


def _mesh():
    return plsc.VectorSubcoreMesh(core_axis_name="c", subcore_axis_name="s")


_LINEAR = pltpu.CompilerParams(use_tc_tiling_on_sc=False)


RB = 104


@functools.partial(
    pl.kernel,
    mesh=_mesh(),
    compiler_params=_LINEAR,
    out_type=jax.ShapeDtypeStruct((2, N, IN_F), jnp.float32),
    scratch_types=[
        pltpu.VMEM((CHUNK, DH), jnp.float32),
        pltpu.VMEM((1, CHUNK), jnp.int32),
        pltpu.VMEM((1, CHUNK), jnp.int32),
        pltpu.VMEM((1, CHUNK), jnp.int32),
        pltpu.VMEM((1, CHUNK), jnp.int32),
        pltpu.VMEM((ROWS_PER_SUB + TAIL_ROWS, DH), jnp.float32),
        pltpu.VMEM((ROWS_PER_SUB + TAIL_ROWS, DH), jnp.float32),
        pltpu.VMEM((RB, IN_F), jnp.float32),
        pltpu.VMEM_SHARED((N, DH), jnp.float32),
        pltpu.VMEM_SHARED((N, DH), jnp.float32),
        pltpu.SemaphoreType.DMA,
        pltpu.SemaphoreType.DMA,
        pltpu.SemaphoreType.DMA,
        pltpu.SemaphoreType.DMA,
    ],
)
def _deg_kernel(src_hbm, dst_hbm, ones_hbm, zeros_hbm, out_hbm,
                ones_v, sidx0, sidx1, didx0, didx1, slab_s, slab_d, wide_v,
                acc_s, acc_d, sem0, sem1, ssem0, ssem1):
    cid = lax.axis_index("c")
    sid = lax.axis_index("s")
    wid = cid * NSUB + sid
    r0 = sid * ROWS_PER_SUB
    sidx = (sidx0, sidx1)
    didx = (didx0, didx1)
    sems = (sem0, sem1)
    ssem = (ssem0, ssem1)
    npair = (ITERS + 1) // 2

    def fire(slot, ordinal, first=False):
        if not first:
            cprev = wid + (ordinal - 2) * NW

            @pl.when(cprev < NCHUNK)
            def _():
                pltpu.make_async_copy(ones_v, acc_s.at[sidx[slot].at[0]],
                                      ssem[slot]).wait()
                pltpu.make_async_copy(ones_v, acc_d.at[didx[slot].at[0]],
                                      ssem[slot]).wait()

        c = wid + ordinal * NW

        @pl.when(c < NCHUNK)
        def _():
            base = c * CHUNK
            pltpu.async_copy(src_hbm.at[pl.ds(base, CHUNK)],
                             sidx[slot].at[0], sems[slot])
            pltpu.async_copy(dst_hbm.at[pl.ds(base, CHUNK)],
                             didx[slot].at[0], sems[slot])

    def drain(slot, ordinal):
        c = wid + ordinal * NW

        @pl.when(c < NCHUNK)
        def _():
            base = c * CHUNK
            pltpu.make_async_copy(src_hbm.at[pl.ds(base, CHUNK)],
                                  sidx[slot].at[0], sems[slot]).wait()
            pltpu.make_async_copy(dst_hbm.at[pl.ds(base, CHUNK)],
                                  didx[slot].at[0], sems[slot]).wait()
            pltpu.async_copy(ones_v, acc_s.at[sidx[slot].at[0]],
                             ssem[slot], add=True)
            pltpu.async_copy(ones_v, acc_d.at[didx[slot].at[0]],
                             ssem[slot], add=True)

    pltpu.sync_copy(ones_hbm, ones_v)
    pltpu.sync_copy(zeros_hbm.at[pl.ds(r0, ROWS_PER_SUB)],
                    acc_s.at[pl.ds(r0, ROWS_PER_SUB)])
    pltpu.sync_copy(zeros_hbm.at[pl.ds(r0, ROWS_PER_SUB)],
                    acc_d.at[pl.ds(r0, ROWS_PER_SUB)])

    @pl.when(sid == NSUB - 1)
    def _():
        pltpu.sync_copy(zeros_hbm.at[pl.ds(NSUB * ROWS_PER_SUB, TAIL_ROWS)],
                        acc_s.at[pl.ds(NSUB * ROWS_PER_SUB, TAIL_ROWS)])
        pltpu.sync_copy(zeros_hbm.at[pl.ds(NSUB * ROWS_PER_SUB, TAIL_ROWS)],
                        acc_d.at[pl.ds(NSUB * ROWS_PER_SUB, TAIL_ROWS)])

    plsc.subcore_barrier()

    fire(0, 0, first=True)
    fire(1, 1, first=True)

    @pl.loop(0, npair)
    def _(j):
        i0 = 2 * j
        drain(0, i0)
        fire(0, i0 + 2)
        drain(1, i0 + 1)
        fire(1, i0 + 3)

    plsc.subcore_barrier()

    def slab_load(acc, slab):
        pltpu.sync_copy(acc.at[pl.ds(r0, ROWS_PER_SUB)],
                        slab.at[pl.ds(0, ROWS_PER_SUB)])

        @pl.when(sid == NSUB - 1)
        def _():
            pltpu.sync_copy(acc.at[pl.ds(NSUB * ROWS_PER_SUB, TAIL_ROWS)],
                            slab.at[pl.ds(ROWS_PER_SUB, TAIL_ROWS)])

    slab_load(acc_s, slab_s)
    slab_load(acc_d, slab_d)

    def expand_rows(off, nrows):
        @pl.loop(0, nrows)
        def _(r):
            vs = slab_s[off + r, :]
            vd = slab_d[off + r, :]
            for k in range(4):
                wide_v[r, pl.ds(k * DH, DH)] = vs
            for k in range(4, 8):
                wide_v[r, pl.ds(k * DH, DH)] = vd

    nblk = ROWS_PER_SUB // RB

    @pl.loop(0, nblk)
    def _(b):
        expand_rows(b * RB, RB)
        pltpu.sync_copy(wide_v, out_hbm.at[cid, pl.ds(r0 + b * RB, RB)])

    @pl.when(sid == NSUB - 1)
    def _():
        expand_rows(ROWS_PER_SUB, TAIL_ROWS)
        pltpu.sync_copy(wide_v.at[pl.ds(0, TAIL_ROWS)],
                        out_hbm.at[cid, pl.ds(NSUB * ROWS_PER_SUB,
                                              TAIL_ROWS)])


def _make_agg(D, nslot, stage_feat=False):
    ngroup = (ITERS + nslot - 1) // nslot

    idx_scratch = [pltpu.VMEM((CHUNK,), jnp.int32) for _ in range(nslot)]
    didx_scratch = [pltpu.VMEM((1, CHUNK), jnp.int32) for _ in range(nslot)]
    row_scratch = [pltpu.VMEM((CHUNK, D), jnp.float32) for _ in range(nslot)]
    sem_scratch = [pltpu.SemaphoreType.DMA for _ in range(2 * nslot)]
    stage_scratch = (
        [pltpu.VMEM_SHARED((N, D), jnp.float32)] if stage_feat else [])

    @functools.partial(
        pl.kernel,
        mesh=_mesh(),
        compiler_params=_LINEAR,
        out_type=jax.ShapeDtypeStruct((2, N, D), jnp.float32),
        scratch_types=idx_scratch + didx_scratch + row_scratch
        + [pltpu.VMEM_SHARED((N, D), jnp.float32)] + sem_scratch
        + stage_scratch,
    )
    def agg(feat_hbm, src_hbm, dst_hbm, zeros_hbm, out_hbm, *scratch):
        sidx = scratch[0:nslot]
        didx = scratch[nslot:2 * nslot]
        rows = scratch[2 * nslot:3 * nslot]
        acc = scratch[3 * nslot]
        gsem = scratch[3 * nslot + 1:3 * nslot + 1 + nslot]
        ssem = scratch[3 * nslot + 1 + nslot:3 * nslot + 1 + 2 * nslot]
        cid = lax.axis_index("c")
        sid = lax.axis_index("s")
        wid = cid * NSUB + sid
        r0 = sid * ROWS_PER_SUB
        feat = scratch[3 * nslot + 1 + 2 * nslot] if stage_feat else feat_hbm

        def fire(slot, ordinal, first=False):
            if not first:
                cprev = wid + (ordinal - nslot) * NW

                @pl.when(cprev < NCHUNK)
                def _():
                    pltpu.make_async_copy(rows[slot],
                                          acc.at[didx[slot].at[0]],
                                          ssem[slot]).wait()

            c = wid + ordinal * NW

            @pl.when(c < NCHUNK)
            def _():
                base = c * CHUNK
                pltpu.sync_copy(src_hbm.at[pl.ds(base, CHUNK)], sidx[slot])
                pltpu.async_copy(feat.at[sidx[slot]], rows[slot], gsem[slot])
                pltpu.sync_copy(dst_hbm.at[pl.ds(base, CHUNK)], didx[slot].at[0])

        def drain(slot, ordinal):
            c = wid + ordinal * NW

            @pl.when(c < NCHUNK)
            def _():
                pltpu.make_async_copy(feat.at[sidx[slot]], rows[slot],
                                      gsem[slot]).wait()
                pltpu.async_copy(rows[slot], acc.at[didx[slot].at[0]],
                                 ssem[slot], add=True)

        pltpu.sync_copy(zeros_hbm.at[pl.ds(r0, ROWS_PER_SUB)],
                        acc.at[pl.ds(r0, ROWS_PER_SUB)])
        if stage_feat:
            pltpu.sync_copy(feat_hbm.at[pl.ds(r0, ROWS_PER_SUB)],
                            feat.at[pl.ds(r0, ROWS_PER_SUB)])

        @pl.when(sid == NSUB - 1)
        def _():
            pltpu.sync_copy(zeros_hbm.at[pl.ds(NSUB * ROWS_PER_SUB, TAIL_ROWS)],
                            acc.at[pl.ds(NSUB * ROWS_PER_SUB, TAIL_ROWS)])
            if stage_feat:
                pltpu.sync_copy(feat_hbm.at[pl.ds(NSUB * ROWS_PER_SUB,
                                                  TAIL_ROWS)],
                                feat.at[pl.ds(NSUB * ROWS_PER_SUB, TAIL_ROWS)])

        plsc.subcore_barrier()

        for s in range(nslot):
            fire(s, s, first=True)

        @pl.loop(0, ngroup)
        def _(j):
            i0 = j * nslot
            for s in range(nslot):
                drain(s, i0 + s)
                fire(s, i0 + s + nslot)

        plsc.subcore_barrier()
        pltpu.sync_copy(acc.at[pl.ds(r0, ROWS_PER_SUB)],
                        out_hbm.at[cid, pl.ds(r0, ROWS_PER_SUB)])

        @pl.when(sid == NSUB - 1)
        def _():
            pltpu.sync_copy(acc.at[pl.ds(NSUB * ROWS_PER_SUB, TAIL_ROWS)],
                            out_hbm.at[cid, pl.ds(NSUB * ROWS_PER_SUB, TAIL_ROWS)])

    return agg


_agg128 = _make_agg(IN_F, 3)
_agg48 = _make_agg(D2, 6)


BN = 2000


def _tc_norm_scale(x, deg_p):
    def body(x_ref, d_ref, xs_ref, nrm_ref):
        d = d_ref[...]
        ns = lax.rsqrt(jnp.maximum(d[0, :, 0:1] + d[1, :, 0:1], 1.0))
        nd = lax.rsqrt(jnp.maximum(d[0, :, 64:65] + d[1, :, 64:65], 1.0))
        nrm_ref[...] = jnp.concatenate([ns, nd], axis=1)
        xs_ref[...] = x_ref[...] * ns

    return pl.pallas_call(
        body,
        grid=(N // BN,),
        in_specs=[
            pl.BlockSpec((BN, IN_F), lambda i: (i, 0)),
            pl.BlockSpec((2, BN, IN_F), lambda i: (0, i, 0)),
        ],
        out_specs=[
            pl.BlockSpec((BN, IN_F), lambda i: (i, 0)),
            pl.BlockSpec((BN, 2), lambda i: (i, 0)),
        ],
        out_shape=[
            jax.ShapeDtypeStruct((N, IN_F), jnp.float32),
            jax.ShapeDtypeStruct((N, 2), jnp.float32),
        ],
    )(x, deg_p)


def _tc_mid(p, nrm, w1, b1, w2, wfc_p):
    def body(p_ref, n_ref, w1_ref, b1_ref, w2_ref, wfc_ref, g_ref):
        nd = n_ref[:, 1]
        ns = n_ref[:, 0]
        a = (p_ref[0] + p_ref[1]) * nd[:, None]
        h = jnp.dot(a, w1_ref[...], preferred_element_type=jnp.float32)
        h = jnp.maximum(h + b1_ref[...][None, :], 0.0)
        wc = jnp.dot(w2_ref[...], wfc_ref[...],
                     preferred_element_type=jnp.float32)
        g_ref[...] = jnp.dot(h * ns[:, None], wc,
                             preferred_element_type=jnp.float32)

    return pl.pallas_call(
        body,
        grid=(N // BN,),
        in_specs=[
            pl.BlockSpec((2, BN, IN_F), lambda i: (0, i, 0)),
            pl.BlockSpec((BN, 2), lambda i: (i, 0)),
            pl.BlockSpec((IN_F, H1F), lambda i: (0, 0)),
            pl.BlockSpec((H1F,), lambda i: (0,)),
            pl.BlockSpec((H1F, H2F), lambda i: (0, 0)),
            pl.BlockSpec((H2F, D2), lambda i: (0, 0)),
        ],
        out_specs=pl.BlockSpec((BN, D2), lambda i: (i, 0)),
        out_shape=jax.ShapeDtypeStruct((N, D2), jnp.float32),
    )(p, nrm, w1, b1, w2, wfc_p)


def _tc_final(q, nrm, b2, wfc_p, bfc_p):
    def body(q_ref, n_ref, b2_ref, wfc_ref, bfc_ref, o_ref):
        bc = jnp.dot(b2_ref[...][None, :], wfc_ref[...],
                     preferred_element_type=jnp.float32)[0] + bfc_ref[...]
        nd = n_ref[:, 1]
        o_ref[...] = (q_ref[0] + q_ref[1]) * nd[:, None] + bc[None, :]

    return pl.pallas_call(
        body,
        grid=(N // BN,),
        in_specs=[
            pl.BlockSpec((2, BN, D2), lambda i: (0, i, 0)),
            pl.BlockSpec((BN, 2), lambda i: (i, 0)),
            pl.BlockSpec((H2F,), lambda i: (0,)),
            pl.BlockSpec((H2F, D2), lambda i: (0, 0)),
            pl.BlockSpec((D2,), lambda i: (0,)),
        ],
        out_specs=pl.BlockSpec((BN, D2), lambda i: (i, 0)),
        out_shape=jax.ShapeDtypeStruct((N, D2), jnp.float32),
    )(q, nrm, b2, wfc_p, bfc_p)


@jax.jit
def kernel(x, edge_index, W1, b1, W2, b2, Wfc, bfc):
    src = edge_index[0]
    dst = edge_index[1]
    ones16 = jnp.ones((CHUNK, DH), jnp.float32)
    zeros16 = jnp.zeros((N, DH), jnp.float32)
    zeros128 = jnp.zeros((N, IN_F), jnp.float32)
    zeros48 = jnp.zeros((N, D2), jnp.float32)
    wfc_p = jnp.pad(Wfc, ((0, 0), (0, D2 - NCLS)))
    bfc_p = jnp.pad(bfc, (0, D2 - NCLS))

    deg_p = _deg_kernel(src, dst, ones16, zeros16)
    xs, nrm = _tc_norm_scale(x, deg_p)
    p = _agg128(xs, src, dst, zeros128)
    g = _tc_mid(p, nrm, W1, b1, W2, wfc_p)
    q = _agg48(g, src, dst, zeros48)
    out = _tc_final(q, nrm, b2, wfc_p, bfc_p)
    return out[:, :NCLS]


# speedup vs baseline: 1.0360x; 1.0001x over previous
import functools

import jax
import jax.numpy as jnp
from jax import lax
from jax.experimental import pallas as pl
from jax.experimental.pallas import tpu as pltpu
from jax.experimental.pallas import tpu_sc as plsc

N = 10000
E = 320000
IN_F = 128
H1F = 256
H2F = 128
NCLS = 47
D2 = 48
DH = 16
CHUNK = 128
NCHUNK = E // CHUNK
NSUB = 16
NW = 2 * NSUB
ITERS = (NCHUNK + NW - 1) // NW
ROWS_PER_SUB = 624
TAIL_ROWS = N - NSUB * ROWS_PER_SUB


def _mesh():
    return plsc.VectorSubcoreMesh(core_axis_name="c", subcore_axis_name="s")


_LINEAR = pltpu.CompilerParams(use_tc_tiling_on_sc=False)


RB = 104


@functools.partial(
    pl.kernel,
    mesh=_mesh(),
    compiler_params=_LINEAR,
    out_type=jax.ShapeDtypeStruct((2, N, IN_F), jnp.float32),
    scratch_types=[
        pltpu.VMEM((CHUNK, DH), jnp.float32),
        pltpu.VMEM((1, CHUNK), jnp.int32),
        pltpu.VMEM((1, CHUNK), jnp.int32),
        pltpu.VMEM((1, CHUNK), jnp.int32),
        pltpu.VMEM((1, CHUNK), jnp.int32),
        pltpu.VMEM((ROWS_PER_SUB + TAIL_ROWS, DH), jnp.float32),
        pltpu.VMEM((ROWS_PER_SUB + TAIL_ROWS, DH), jnp.float32),
        pltpu.VMEM((RB, IN_F), jnp.float32),
        pltpu.VMEM_SHARED((N, DH), jnp.float32),
        pltpu.VMEM_SHARED((N, DH), jnp.float32),
        pltpu.SemaphoreType.DMA,
        pltpu.SemaphoreType.DMA,
        pltpu.SemaphoreType.DMA,
        pltpu.SemaphoreType.DMA,
    ],
)
def _deg_kernel(src_hbm, dst_hbm, ones_hbm, zeros_hbm, out_hbm,
                ones_v, sidx0, sidx1, didx0, didx1, slab_s, slab_d, wide_v,
                acc_s, acc_d, sem0, sem1, ssem0, ssem1):
    cid = lax.axis_index("c")
    sid = lax.axis_index("s")
    wid = cid * NSUB + sid
    r0 = sid * ROWS_PER_SUB
    sidx = (sidx0, sidx1)
    didx = (didx0, didx1)
    sems = (sem0, sem1)
    ssem = (ssem0, ssem1)
    npair = (ITERS + 1) // 2

    def fire(slot, ordinal, first=False):
        if not first:
            cprev = wid + (ordinal - 2) * NW

            @pl.when(cprev < NCHUNK)
            def _():
                pltpu.make_async_copy(ones_v, acc_s.at[sidx[slot].at[0]],
                                      ssem[slot]).wait()
                pltpu.make_async_copy(ones_v, acc_d.at[didx[slot].at[0]],
                                      ssem[slot]).wait()

        c = wid + ordinal * NW

        @pl.when(c < NCHUNK)
        def _():
            base = c * CHUNK
            pltpu.async_copy(src_hbm.at[pl.ds(base, CHUNK)],
                             sidx[slot].at[0], sems[slot])
            pltpu.async_copy(dst_hbm.at[pl.ds(base, CHUNK)],
                             didx[slot].at[0], sems[slot])

    def drain(slot, ordinal):
        c = wid + ordinal * NW

        @pl.when(c < NCHUNK)
        def _():
            base = c * CHUNK
            pltpu.make_async_copy(src_hbm.at[pl.ds(base, CHUNK)],
                                  sidx[slot].at[0], sems[slot]).wait()
            pltpu.make_async_copy(dst_hbm.at[pl.ds(base, CHUNK)],
                                  didx[slot].at[0], sems[slot]).wait()
            pltpu.async_copy(ones_v, acc_s.at[sidx[slot].at[0]],
                             ssem[slot], add=True)
            pltpu.async_copy(ones_v, acc_d.at[didx[slot].at[0]],
                             ssem[slot], add=True)

    pltpu.sync_copy(ones_hbm, ones_v)
    pltpu.sync_copy(zeros_hbm.at[pl.ds(r0, ROWS_PER_SUB)],
                    acc_s.at[pl.ds(r0, ROWS_PER_SUB)])
    pltpu.sync_copy(zeros_hbm.at[pl.ds(r0, ROWS_PER_SUB)],
                    acc_d.at[pl.ds(r0, ROWS_PER_SUB)])

    @pl.when(sid == NSUB - 1)
    def _():
        pltpu.sync_copy(zeros_hbm.at[pl.ds(NSUB * ROWS_PER_SUB, TAIL_ROWS)],
                        acc_s.at[pl.ds(NSUB * ROWS_PER_SUB, TAIL_ROWS)])
        pltpu.sync_copy(zeros_hbm.at[pl.ds(NSUB * ROWS_PER_SUB, TAIL_ROWS)],
                        acc_d.at[pl.ds(NSUB * ROWS_PER_SUB, TAIL_ROWS)])

    plsc.subcore_barrier()

    fire(0, 0, first=True)
    fire(1, 1, first=True)

    @pl.loop(0, npair)
    def _(j):
        i0 = 2 * j
        drain(0, i0)
        fire(0, i0 + 2)
        drain(1, i0 + 1)
        fire(1, i0 + 3)

    plsc.subcore_barrier()

    def slab_load(acc, slab):
        pltpu.sync_copy(acc.at[pl.ds(r0, ROWS_PER_SUB)],
                        slab.at[pl.ds(0, ROWS_PER_SUB)])

        @pl.when(sid == NSUB - 1)
        def _():
            pltpu.sync_copy(acc.at[pl.ds(NSUB * ROWS_PER_SUB, TAIL_ROWS)],
                            slab.at[pl.ds(ROWS_PER_SUB, TAIL_ROWS)])

    slab_load(acc_s, slab_s)
    slab_load(acc_d, slab_d)

    def expand_rows(off, nrows):
        @pl.loop(0, nrows)
        def _(r):
            vs = slab_s[off + r, :]
            vd = slab_d[off + r, :]
            for k in range(4):
                wide_v[r, pl.ds(k * DH, DH)] = vs
            for k in range(4, 8):
                wide_v[r, pl.ds(k * DH, DH)] = vd

    nblk = ROWS_PER_SUB // RB

    @pl.loop(0, nblk)
    def _(b):
        expand_rows(b * RB, RB)
        pltpu.sync_copy(wide_v, out_hbm.at[cid, pl.ds(r0 + b * RB, RB)])

    @pl.when(sid == NSUB - 1)
    def _():
        expand_rows(ROWS_PER_SUB, TAIL_ROWS)
        pltpu.sync_copy(wide_v.at[pl.ds(0, TAIL_ROWS)],
                        out_hbm.at[cid, pl.ds(NSUB * ROWS_PER_SUB,
                                              TAIL_ROWS)])


def _make_agg(D, nslot, stage_feat=False):
    ngroup = (ITERS + nslot - 1) // nslot

    idx_scratch = [pltpu.VMEM((CHUNK,), jnp.int32) for _ in range(nslot)]
    didx_scratch = [pltpu.VMEM((1, CHUNK), jnp.int32) for _ in range(nslot)]
    row_scratch = [pltpu.VMEM((CHUNK, D), jnp.float32) for _ in range(nslot)]
    sem_scratch = [pltpu.SemaphoreType.DMA for _ in range(2 * nslot)]
    stage_scratch = (
        [pltpu.VMEM_SHARED((N, D), jnp.float32)] if stage_feat else [])

    @functools.partial(
        pl.kernel,
        mesh=_mesh(),
        compiler_params=_LINEAR,
        out_type=jax.ShapeDtypeStruct((2, N, D), jnp.float32),
        scratch_types=idx_scratch + didx_scratch + row_scratch
        + [pltpu.VMEM_SHARED((N, D), jnp.float32)] + sem_scratch
        + stage_scratch,
    )
    def agg(feat_hbm, src_hbm, dst_hbm, zeros_hbm, out_hbm, *scratch):
        sidx = scratch[0:nslot]
        didx = scratch[nslot:2 * nslot]
        rows = scratch[2 * nslot:3 * nslot]
        acc = scratch[3 * nslot]
        gsem = scratch[3 * nslot + 1:3 * nslot + 1 + nslot]
        ssem = scratch[3 * nslot + 1 + nslot:3 * nslot + 1 + 2 * nslot]
        cid = lax.axis_index("c")
        sid = lax.axis_index("s")
        wid = cid * NSUB + sid
        r0 = sid * ROWS_PER_SUB
        feat = scratch[3 * nslot + 1 + 2 * nslot] if stage_feat else feat_hbm

        def fire(slot, ordinal, first=False):
            if not first:
                cprev = wid + (ordinal - nslot) * NW

                @pl.when(cprev < NCHUNK)
                def _():
                    pltpu.make_async_copy(rows[slot],
                                          acc.at[didx[slot].at[0]],
                                          ssem[slot]).wait()

            c = wid + ordinal * NW

            @pl.when(c < NCHUNK)
            def _():
                base = c * CHUNK
                pltpu.sync_copy(src_hbm.at[pl.ds(base, CHUNK)], sidx[slot])
                pltpu.async_copy(feat.at[sidx[slot]], rows[slot], gsem[slot])
                pltpu.sync_copy(dst_hbm.at[pl.ds(base, CHUNK)], didx[slot].at[0])

        def drain(slot, ordinal):
            c = wid + ordinal * NW

            @pl.when(c < NCHUNK)
            def _():
                pltpu.make_async_copy(feat.at[sidx[slot]], rows[slot],
                                      gsem[slot]).wait()
                pltpu.async_copy(rows[slot], acc.at[didx[slot].at[0]],
                                 ssem[slot], add=True)

        pltpu.sync_copy(zeros_hbm.at[pl.ds(r0, ROWS_PER_SUB)],
                        acc.at[pl.ds(r0, ROWS_PER_SUB)])
        if stage_feat:
            pltpu.sync_copy(feat_hbm.at[pl.ds(r0, ROWS_PER_SUB)],
                            feat.at[pl.ds(r0, ROWS_PER_SUB)])

        @pl.when(sid == NSUB - 1)
        def _():
            pltpu.sync_copy(zeros_hbm.at[pl.ds(NSUB * ROWS_PER_SUB, TAIL_ROWS)],
                            acc.at[pl.ds(NSUB * ROWS_PER_SUB, TAIL_ROWS)])
            if stage_feat:
                pltpu.sync_copy(feat_hbm.at[pl.ds(NSUB * ROWS_PER_SUB,
                                                  TAIL_ROWS)],
                                feat.at[pl.ds(NSUB * ROWS_PER_SUB, TAIL_ROWS)])

        plsc.subcore_barrier()

        for s in range(nslot):
            fire(s, s, first=True)

        @pl.loop(0, ngroup)
        def _(j):
            i0 = j * nslot
            for s in range(nslot):
                drain(s, i0 + s)
                fire(s, i0 + s + nslot)

        plsc.subcore_barrier()
        pltpu.sync_copy(acc.at[pl.ds(r0, ROWS_PER_SUB)],
                        out_hbm.at[cid, pl.ds(r0, ROWS_PER_SUB)])

        @pl.when(sid == NSUB - 1)
        def _():
            pltpu.sync_copy(acc.at[pl.ds(NSUB * ROWS_PER_SUB, TAIL_ROWS)],
                            out_hbm.at[cid, pl.ds(NSUB * ROWS_PER_SUB, TAIL_ROWS)])

    return agg


_agg128 = _make_agg(IN_F, 3)
_agg48 = _make_agg(D2, 6)


BN = 2000


def _tc_norm_scale(x, deg_p):
    def body(x_ref, d_ref, xs_ref, nrm_ref):
        d = d_ref[...]
        ns = lax.rsqrt(jnp.maximum(d[0, :, 0:1] + d[1, :, 0:1], 1.0))
        nd = lax.rsqrt(jnp.maximum(d[0, :, 64:65] + d[1, :, 64:65], 1.0))
        nrm_ref[...] = jnp.concatenate([ns, nd], axis=1)
        xs_ref[...] = x_ref[...] * ns

    return pl.pallas_call(
        body,
        grid=(N // BN,),
        in_specs=[
            pl.BlockSpec((BN, IN_F), lambda i: (i, 0)),
            pl.BlockSpec((2, BN, IN_F), lambda i: (0, i, 0)),
        ],
        out_specs=[
            pl.BlockSpec((BN, IN_F), lambda i: (i, 0)),
            pl.BlockSpec((BN, 2), lambda i: (i, 0)),
        ],
        out_shape=[
            jax.ShapeDtypeStruct((N, IN_F), jnp.float32),
            jax.ShapeDtypeStruct((N, 2), jnp.float32),
        ],
    )(x, deg_p)


def _tc_mid(p, nrm, w1, b1, w2, wfc_p):
    def body(p_ref, n_ref, w1_ref, b1_ref, w2_ref, wfc_ref, g_ref):
        nd = n_ref[:, 1]
        ns = n_ref[:, 0]
        a = (p_ref[0] + p_ref[1]) * nd[:, None]
        h = jnp.dot(a, w1_ref[...], preferred_element_type=jnp.float32)
        h = jnp.maximum(h + b1_ref[...][None, :], 0.0)
        wc = jnp.dot(w2_ref[...], wfc_ref[...],
                     preferred_element_type=jnp.float32)
        g_ref[...] = jnp.dot(h * ns[:, None], wc,
                             preferred_element_type=jnp.float32)

    return pl.pallas_call(
        body,
        grid=(N // BN,),
        in_specs=[
            pl.BlockSpec((2, BN, IN_F), lambda i: (0, i, 0)),
            pl.BlockSpec((BN, 2), lambda i: (i, 0)),
            pl.BlockSpec((IN_F, H1F), lambda i: (0, 0)),
            pl.BlockSpec((H1F,), lambda i: (0,)),
            pl.BlockSpec((H1F, H2F), lambda i: (0, 0)),
            pl.BlockSpec((H2F, D2), lambda i: (0, 0)),
        ],
        out_specs=pl.BlockSpec((BN, D2), lambda i: (i, 0)),
        out_shape=jax.ShapeDtypeStruct((N, D2), jnp.float32),
    )(p, nrm, w1, b1, w2, wfc_p)


def _tc_final(q, nrm, b2, wfc_p, bfc_p):
    def body(q_ref, n_ref, b2_ref, wfc_ref, bfc_ref, o_ref):
        bc = jnp.dot(b2_ref[...][None, :], wfc_ref[...],
                     preferred_element_type=jnp.float32)[0] + bfc_ref[...]
        nd = n_ref[:, 1]
        full = (q_ref[0] + q_ref[1]) * nd[:, None] + bc[None, :]
        o_ref[...] = full[:, :NCLS]

    return pl.pallas_call(
        body,
        grid=(N // BN,),
        in_specs=[
            pl.BlockSpec((2, BN, D2), lambda i: (0, i, 0)),
            pl.BlockSpec((BN, 2), lambda i: (i, 0)),
            pl.BlockSpec((H2F,), lambda i: (0,)),
            pl.BlockSpec((H2F, D2), lambda i: (0, 0)),
            pl.BlockSpec((D2,), lambda i: (0,)),
        ],
        out_specs=pl.BlockSpec((BN, NCLS), lambda i: (i, 0)),
        out_shape=jax.ShapeDtypeStruct((N, NCLS), jnp.float32),
    )(q, nrm, b2, wfc_p, bfc_p)


@jax.jit
def kernel(x, edge_index, W1, b1, W2, b2, Wfc, bfc):
    src = edge_index[0]
    dst = edge_index[1]
    ones16 = jnp.ones((CHUNK, DH), jnp.float32)
    zeros16 = jnp.zeros((N, DH), jnp.float32)
    zeros128 = jnp.zeros((N, IN_F), jnp.float32)
    zeros48 = jnp.zeros((N, D2), jnp.float32)
    wfc_p = jnp.pad(Wfc, ((0, 0), (0, D2 - NCLS)))
    bfc_p = jnp.pad(bfc, (0, D2 - NCLS))

    deg_p = _deg_kernel(src, dst, ones16, zeros16)
    xs, nrm = _tc_norm_scale(x, deg_p)
    p = _agg128(xs, src, dst, zeros128)
    g = _tc_mid(p, nrm, W1, b1, W2, wfc_p)
    q = _agg48(g, src, dst, zeros48)
    return _tc_final(q, nrm, b2, wfc_p, bfc_p)


# speedup vs baseline: 1.2929x; 1.2479x over previous
import functools

import jax
import jax.numpy as jnp
from jax import lax
from jax.experimental import pallas as pl
from jax.experimental.pallas import tpu as pltpu
from jax.experimental.pallas import tpu_sc as plsc

N = 10000
E = 320000
IN_F = 128
H1F = 256
H2F = 128
NCLS = 47
D2 = 48
DH = 16
CHUNK = 128
NCHUNK = E // CHUNK
NSUB = 16
NW = 2 * NSUB
ITERS = (NCHUNK + NW - 1) // NW
ROWS_PER_SUB = 624
TAIL_ROWS = N - NSUB * ROWS_PER_SUB
NCHUNK_PAD = 2512
CPW = NCHUNK // NW


def _mesh():
    return plsc.VectorSubcoreMesh(core_axis_name="c", subcore_axis_name="s")


_LINEAR = pltpu.CompilerParams(use_tc_tiling_on_sc=False)


RB = 104


@functools.partial(
    pl.kernel,
    mesh=_mesh(),
    compiler_params=_LINEAR,
    out_type=jax.ShapeDtypeStruct((2, N, IN_F), jnp.float32),
    scratch_types=[
        pltpu.VMEM((CHUNK, DH), jnp.float32),
        pltpu.VMEM((2, CHUNK), jnp.int32),
        pltpu.VMEM((2, CHUNK), jnp.int32),
        pltpu.VMEM((ROWS_PER_SUB + TAIL_ROWS, DH), jnp.float32),
        pltpu.VMEM((ROWS_PER_SUB + TAIL_ROWS, DH), jnp.float32),
        pltpu.VMEM((RB, IN_F), jnp.float32),
        pltpu.VMEM_SHARED((N, DH), jnp.float32),
        pltpu.VMEM_SHARED((N, DH), jnp.float32),
        pltpu.SemaphoreType.DMA,
        pltpu.SemaphoreType.DMA,
        pltpu.SemaphoreType.DMA,
        pltpu.SemaphoreType.DMA,
    ],
)
def _deg_kernel(sd_hbm, ones_hbm, zeros_hbm, out_hbm,
                ones_v, sd0, sd1, slab_s, slab_d, wide_v,
                acc_s, acc_d, sem0, sem1, ssem0, ssem1):
    cid = lax.axis_index("c")
    sid = lax.axis_index("s")
    wid = cid * NSUB + sid
    r0 = sid * ROWS_PER_SUB
    sdv = (sd0, sd1)
    sems = (sem0, sem1)
    ssem = (ssem0, ssem1)
    npair = (ITERS + 1) // 2

    def fire(slot, ordinal, first=False):
        if not first:
            cprev = wid + (ordinal - 2) * NW

            @pl.when(cprev < NCHUNK)
            def _():
                pltpu.make_async_copy(ones_v, acc_s.at[sdv[slot].at[0]],
                                      ssem[slot]).wait()
                pltpu.make_async_copy(ones_v, acc_d.at[sdv[slot].at[1]],
                                      ssem[slot]).wait()

        c = wid + ordinal * NW

        @pl.when(c < NCHUNK)
        def _():
            pltpu.async_copy(sd_hbm.at[c], sdv[slot], sems[slot])

    def drain(slot, ordinal):
        c = wid + ordinal * NW

        @pl.when(c < NCHUNK)
        def _():
            pltpu.make_async_copy(sd_hbm.at[c], sdv[slot], sems[slot]).wait()
            pltpu.async_copy(ones_v, acc_s.at[sdv[slot].at[0]],
                             ssem[slot], add=True)
            pltpu.async_copy(ones_v, acc_d.at[sdv[slot].at[1]],
                             ssem[slot], add=True)

    pltpu.sync_copy(ones_hbm, ones_v)
    pltpu.sync_copy(zeros_hbm.at[pl.ds(r0, ROWS_PER_SUB)],
                    acc_s.at[pl.ds(r0, ROWS_PER_SUB)])
    pltpu.sync_copy(zeros_hbm.at[pl.ds(r0, ROWS_PER_SUB)],
                    acc_d.at[pl.ds(r0, ROWS_PER_SUB)])

    @pl.when(sid == NSUB - 1)
    def _():
        pltpu.sync_copy(zeros_hbm.at[pl.ds(NSUB * ROWS_PER_SUB, TAIL_ROWS)],
                        acc_s.at[pl.ds(NSUB * ROWS_PER_SUB, TAIL_ROWS)])
        pltpu.sync_copy(zeros_hbm.at[pl.ds(NSUB * ROWS_PER_SUB, TAIL_ROWS)],
                        acc_d.at[pl.ds(NSUB * ROWS_PER_SUB, TAIL_ROWS)])

    plsc.subcore_barrier()

    fire(0, 0, first=True)
    fire(1, 1, first=True)

    @pl.loop(0, npair)
    def _(j):
        i0 = 2 * j
        drain(0, i0)
        fire(0, i0 + 2)
        drain(1, i0 + 1)
        fire(1, i0 + 3)

    plsc.subcore_barrier()

    def slab_load(acc, slab):
        pltpu.sync_copy(acc.at[pl.ds(r0, ROWS_PER_SUB)],
                        slab.at[pl.ds(0, ROWS_PER_SUB)])

        @pl.when(sid == NSUB - 1)
        def _():
            pltpu.sync_copy(acc.at[pl.ds(NSUB * ROWS_PER_SUB, TAIL_ROWS)],
                            slab.at[pl.ds(ROWS_PER_SUB, TAIL_ROWS)])

    slab_load(acc_s, slab_s)
    slab_load(acc_d, slab_d)

    def expand_rows(off, nrows):
        @pl.loop(0, nrows)
        def _(r):
            vs = slab_s[off + r, :]
            vd = slab_d[off + r, :]
            for k in range(4):
                wide_v[r, pl.ds(k * DH, DH)] = vs
            for k in range(4, 8):
                wide_v[r, pl.ds(k * DH, DH)] = vd

    nblk = ROWS_PER_SUB // RB

    @pl.loop(0, nblk)
    def _(b):
        expand_rows(b * RB, RB)
        pltpu.sync_copy(wide_v, out_hbm.at[cid, pl.ds(r0 + b * RB, RB)])

    @pl.when(sid == NSUB - 1)
    def _():
        expand_rows(ROWS_PER_SUB, TAIL_ROWS)
        pltpu.sync_copy(wide_v.at[pl.ds(0, TAIL_ROWS)],
                        out_hbm.at[cid, pl.ds(NSUB * ROWS_PER_SUB,
                                              TAIL_ROWS)])


def _make_agg(D, nslot, preload=False):
    ngroup = (ITERS + nslot - 1) // nslot

    if preload:
        idx_scratch = [pltpu.VMEM((CPW + 1, 2, CHUNK), jnp.int32)]
    else:
        idx_scratch = [pltpu.VMEM((2, CHUNK), jnp.int32)
                       for _ in range(nslot)]
    row_scratch = [pltpu.VMEM((CHUNK, D), jnp.float32) for _ in range(nslot)]
    sem_scratch = [pltpu.SemaphoreType.DMA for _ in range(2 * nslot)]

    @functools.partial(
        pl.kernel,
        mesh=_mesh(),
        compiler_params=_LINEAR,
        out_type=jax.ShapeDtypeStruct((2, N, D), jnp.float32),
        scratch_types=idx_scratch + row_scratch
        + [pltpu.VMEM_SHARED((N, D), jnp.float32)] + sem_scratch,
    )
    def agg(feat_hbm, sd_hbm, zeros_hbm, out_hbm, *scratch):
        nidx = 1 if preload else nslot
        idxs = scratch[0:nidx]
        rows = scratch[nidx:nidx + nslot]
        acc = scratch[nidx + nslot]
        gsem = scratch[nidx + nslot + 1:nidx + 2 * nslot + 1]
        ssem = scratch[nidx + 2 * nslot + 1:]
        cid = lax.axis_index("c")
        sid = lax.axis_index("s")
        wid = cid * NSUB + sid
        r0 = sid * ROWS_PER_SUB
        if preload:
            start = wid * CPW + jnp.minimum(wid, NCHUNK - NW * CPW)
            cnt = CPW + (wid < NCHUNK - NW * CPW)

            def sd_at(slot, ordinal):
                return idxs[0].at[ordinal]

            def valid(ordinal):
                return ordinal < cnt
        else:
            def sd_at(slot, ordinal):
                return idxs[slot]

            def valid(ordinal):
                return wid + ordinal * NW < NCHUNK

        def fire(slot, ordinal, first=False):
            if not first:
                @pl.when(valid(ordinal - nslot))
                def _():
                    pltpu.make_async_copy(
                        rows[slot],
                        acc.at[sd_at(slot, ordinal - nslot).at[1]],
                        ssem[slot]).wait()

            @pl.when(valid(ordinal))
            def _():
                if not preload:
                    pltpu.sync_copy(sd_hbm.at[wid + ordinal * NW], idxs[slot])
                pltpu.async_copy(feat_hbm.at[sd_at(slot, ordinal).at[0]],
                                 rows[slot], gsem[slot])

        def drain(slot, ordinal):
            @pl.when(valid(ordinal))
            def _():
                pltpu.make_async_copy(feat_hbm.at[sd_at(slot, ordinal).at[0]],
                                      rows[slot], gsem[slot]).wait()
                pltpu.async_copy(rows[slot],
                                 acc.at[sd_at(slot, ordinal).at[1]],
                                 ssem[slot], add=True)

        pltpu.sync_copy(zeros_hbm.at[pl.ds(r0, ROWS_PER_SUB)],
                        acc.at[pl.ds(r0, ROWS_PER_SUB)])
        if preload:
            pltpu.sync_copy(sd_hbm.at[pl.ds(start, CPW + 1)], idxs[0])

        @pl.when(sid == NSUB - 1)
        def _():
            pltpu.sync_copy(zeros_hbm.at[pl.ds(NSUB * ROWS_PER_SUB, TAIL_ROWS)],
                            acc.at[pl.ds(NSUB * ROWS_PER_SUB, TAIL_ROWS)])

        plsc.subcore_barrier()

        for s in range(nslot):
            fire(s, s, first=True)

        @pl.loop(0, ngroup)
        def _(j):
            i0 = j * nslot
            for s in range(nslot):
                drain(s, i0 + s)
                fire(s, i0 + s + nslot)

        plsc.subcore_barrier()
        pltpu.sync_copy(acc.at[pl.ds(r0, ROWS_PER_SUB)],
                        out_hbm.at[cid, pl.ds(r0, ROWS_PER_SUB)])

        @pl.when(sid == NSUB - 1)
        def _():
            pltpu.sync_copy(acc.at[pl.ds(NSUB * ROWS_PER_SUB, TAIL_ROWS)],
                            out_hbm.at[cid, pl.ds(NSUB * ROWS_PER_SUB, TAIL_ROWS)])

    return agg


_agg128 = _make_agg(IN_F, 3)
_agg48 = _make_agg(D2, 6)


BN = 2000


def _tc_norm_scale(x, deg_p):
    def body(x_ref, d_ref, xs_ref, nrm_ref):
        d = d_ref[...]
        ns = lax.rsqrt(jnp.maximum(d[0, :, 0:1] + d[1, :, 0:1], 1.0))
        nd = lax.rsqrt(jnp.maximum(d[0, :, 64:65] + d[1, :, 64:65], 1.0))
        nrm_ref[...] = jnp.concatenate([ns, nd], axis=1)
        xs_ref[...] = x_ref[...] * ns

    return pl.pallas_call(
        body,
        grid=(N // BN,),
        in_specs=[
            pl.BlockSpec((BN, IN_F), lambda i: (i, 0)),
            pl.BlockSpec((2, BN, IN_F), lambda i: (0, i, 0)),
        ],
        out_specs=[
            pl.BlockSpec((BN, IN_F), lambda i: (i, 0)),
            pl.BlockSpec((BN, 2), lambda i: (i, 0)),
        ],
        out_shape=[
            jax.ShapeDtypeStruct((N, IN_F), jnp.float32),
            jax.ShapeDtypeStruct((N, 2), jnp.float32),
        ],
    )(x, deg_p)


def _tc_mid(p, nrm, w1, b1, w2, wfc_p):
    def body(p_ref, n_ref, w1_ref, b1_ref, w2_ref, wfc_ref, g_ref):
        nd = n_ref[:, 1]
        ns = n_ref[:, 0]
        a = (p_ref[0] + p_ref[1]) * nd[:, None]
        h = jnp.dot(a, w1_ref[...], preferred_element_type=jnp.float32)
        h = jnp.maximum(h + b1_ref[...][None, :], 0.0)
        wc = jnp.dot(w2_ref[...], wfc_ref[...],
                     preferred_element_type=jnp.float32)
        g_ref[...] = jnp.dot(h * ns[:, None], wc,
                             preferred_element_type=jnp.float32)

    return pl.pallas_call(
        body,
        grid=(N // BN,),
        in_specs=[
            pl.BlockSpec((2, BN, IN_F), lambda i: (0, i, 0)),
            pl.BlockSpec((BN, 2), lambda i: (i, 0)),
            pl.BlockSpec((IN_F, H1F), lambda i: (0, 0)),
            pl.BlockSpec((H1F,), lambda i: (0,)),
            pl.BlockSpec((H1F, H2F), lambda i: (0, 0)),
            pl.BlockSpec((H2F, D2), lambda i: (0, 0)),
        ],
        out_specs=pl.BlockSpec((BN, D2), lambda i: (i, 0)),
        out_shape=jax.ShapeDtypeStruct((N, D2), jnp.float32),
    )(p, nrm, w1, b1, w2, wfc_p)


def _tc_final(q, nrm, b2, wfc_p, bfc_p):
    def body(q_ref, n_ref, b2_ref, wfc_ref, bfc_ref, o_ref):
        bc = jnp.dot(b2_ref[...][None, :], wfc_ref[...],
                     preferred_element_type=jnp.float32)[0] + bfc_ref[...]
        nd = n_ref[:, 1]
        full = (q_ref[0] + q_ref[1]) * nd[:, None] + bc[None, :]
        o_ref[...] = full[:, :NCLS]

    return pl.pallas_call(
        body,
        grid=(N // BN,),
        in_specs=[
            pl.BlockSpec((2, BN, D2), lambda i: (0, i, 0)),
            pl.BlockSpec((BN, 2), lambda i: (i, 0)),
            pl.BlockSpec((H2F,), lambda i: (0,)),
            pl.BlockSpec((H2F, D2), lambda i: (0, 0)),
            pl.BlockSpec((D2,), lambda i: (0,)),
        ],
        out_specs=pl.BlockSpec((BN, NCLS), lambda i: (i, 0)),
        out_shape=jax.ShapeDtypeStruct((N, NCLS), jnp.float32),
    )(q, nrm, b2, wfc_p, bfc_p)


@jax.jit
def kernel(x, edge_index, W1, b1, W2, b2, Wfc, bfc):
    sd = edge_index.reshape(2, NCHUNK, CHUNK).transpose(1, 0, 2)
    ones16 = jnp.ones((CHUNK, DH), jnp.float32)
    zeros16 = jnp.zeros((N, DH), jnp.float32)
    zeros128 = jnp.zeros((N, IN_F), jnp.float32)
    zeros48 = jnp.zeros((N, D2), jnp.float32)
    wfc_p = jnp.pad(Wfc, ((0, 0), (0, D2 - NCLS)))
    bfc_p = jnp.pad(bfc, (0, D2 - NCLS))

    deg_p = _deg_kernel(sd, ones16, zeros16)
    xs, nrm = _tc_norm_scale(x, deg_p)
    p = _agg128(xs, sd, zeros128)
    g = _tc_mid(p, nrm, W1, b1, W2, wfc_p)
    q = _agg48(g, sd, zeros48)
    return _tc_final(q, nrm, b2, wfc_p, bfc_p)
